# baseline (speedup 1.0000x reference)
import math

import jax
import jax.numpy as jnp
import numpy as np
from jax import lax
from jax.experimental import pallas as pl
from jax.experimental.pallas import tpu as pltpu
from jax.experimental.pallas import tpu_sc as plsc

HIDDEN = 512
PAD_MULT = 128
EXTRA = 1
MAX_LEN = 2176
LANES = 16
CH = 32
BW = 48

_SCHED_LENGTHS = (2048, 512, 1024, 1536, 768, 1280, 896, 1152,
                  640, 1408, 1024, 704, 960, 832, 1088, 512)


def _sinusoidal_pos_emb(max_len: int, d: int) -> np.ndarray:
    pos = np.arange(max_len, dtype=np.float32)[:, None]
    i = np.arange(0, d, 2, dtype=np.float32)
    div = np.exp(-math.log(10000.0) * i / d)
    pe = np.zeros((max_len, d), dtype=np.float32)
    pe[:, 0::2] = np.sin(pos * div)
    pe[:, 1::2] = np.cos(pos * div)
    return pe


def _chunk_schedule(n_chunks: int, n_workers: int) -> np.ndarray:
    weights = []
    for c in range(n_chunks):
        g = sum(1 for L in _SCHED_LENGTHS if c * CH <= L)
        weights.append(16 + 2 * g)
    order = sorted(range(n_chunks), key=lambda c: -weights[c])
    loads = [0] * n_workers
    bins = [[] for _ in range(n_workers)]
    for c in order:
        w = min(range(n_workers), key=lambda i: (loads[i], len(bins[i])))
        bins[w].append(c)
        loads[w] += weights[c]
    k = max(len(b) for b in bins)
    sched = np.full((n_workers, 1, k + LANES), n_chunks, dtype=np.int32)
    for w, b in enumerate(bins):
        sched[w, 0, :len(b)] = b
    return sched


def kernel(input_embs, input_seq_lengths, beg_seq_param):
    T, D = input_embs.shape
    B = input_seq_lengths.shape[0]
    ML = MAX_LEN
    scale = jnp.float32(math.sqrt(D))

    mesh = plsc.VectorSubcoreMesh(core_axis_name="c", subcore_axis_name="s")
    NC, NS = mesh.num_cores, mesh.num_subcores
    NW = NC * NS
    n_chunks = ML // CH
    assert ML % CH == 0
    NV = D // LANES

    pe_tab = jnp.asarray(_sinusoidal_pos_emb(ML, D))
    sched_tab = jnp.asarray(_chunk_schedule(n_chunks, NW))
    SW = sched_tab.shape[2]
    K = SW - LANES

    lengths = input_seq_lengths.astype(jnp.int32)
    starts = jnp.concatenate([jnp.zeros((1,), jnp.int32),
                              jnp.cumsum(lengths)[:-1].astype(jnp.int32)])
    cond = (CH * jnp.arange(n_chunks, dtype=jnp.int32)[:, None]) <= lengths[None, :]
    gcnt = jnp.sum(cond, axis=1).astype(jnp.int32)
    ordr = jnp.argsort(jnp.logical_not(cond), axis=1, stable=True).astype(jnp.int32)
    desc = jnp.concatenate(
        [gcnt[:, None], gcnt[:, None], ordr,
         jnp.zeros((n_chunks, BW - 2 - B), jnp.int32)], axis=1)
    dummy = jnp.concatenate(
        [jnp.zeros((1, 1), jnp.int32), jnp.full((1, 1), B, jnp.int32),
         jnp.zeros((1, BW - 2), jnp.int32)], axis=1)
    desc = jnp.concatenate([desc, dummy], axis=0)[:, None, :]

    def body(x_hbm, len_hbm, st_hbm, beg_hbm, pe_hbm, sched_hbm, bl_hbm,
             out_hbm,
             len_v, st_v, pe0_v, pe1_v, xa_v, xb_v, idxa_v, idxb_v,
             bl0_v, bl1_v, schv, beg_v,
             sg_a, sg_b, ss_a, ss_b, s_pe, s_lp0, s_lp1, s_lb0, s_lb1):
        cid = lax.axis_index("c")
        sid = lax.axis_index("s")
        w = sid * NC + cid

        pltpu.sync_copy(len_hbm, len_v.at[pl.ds(0, B)])
        pltpu.sync_copy(st_hbm, st_v.at[pl.ds(0, B)])
        pltpu.sync_copy(sched_hbm.at[w, 0], schv)

        iota = lax.iota(jnp.int32, LANES)

        pltpu.sync_copy(beg_hbm, beg_v)
        for k in range(NV):
            sl = pl.ds(k * LANES, LANES)
            beg_v[sl] = beg_v[sl] * scale

        gbufs = ((xa_v, idxa_v, sg_a, ss_a), (xb_v, idxb_v, sg_b, ss_b))
        pebufs = (pe0_v, pe1_v)
        blbufs = (bl0_v, bl1_v)
        lpsems = (s_lp0, s_lp1)
        lbsems = (s_lb0, s_lb1)

        def slot_c(j):
            return schv[pl.ds(j, LANES)][0]

        def slot_p0(j):
            return jnp.minimum(slot_c(j), n_chunks - 1) * CH

        def issue_slot_loads(j, par):
            c = slot_c(j)
            pltpu.async_copy(pe_hbm.at[pl.ds(slot_p0(j), CH)], pebufs[par],
                             lpsems[par])
            pltpu.async_copy(bl_hbm.at[c, 0], blbufs[par], lbsems[par])

        def get_b(bl_v, i):
            return bl_v[pl.ds(i + 2, LANES)][0]

        def issue_gather(bl_v, p0, i, gpar):
            x_v, idx_v, sg, _ = gbufs[gpar]
            b = get_b(bl_v, i)
            st_b = st_v[pl.ds(b, LANES)][0]
            base = st_b + p0 - 1
            for j in range(CH // LANES):
                idx_v[pl.ds(j * LANES, LANES)] = jnp.clip(
                    base + j * LANES + iota, 0, T - 1)
            pltpu.async_copy(x_hbm.at[idx_v], x_v, sg)

        def compute_store(bl_v, pe_v, p0, i, gpar):
            x_v, idx_v, sg, ss = gbufs[gpar]
            b = get_b(bl_v, i)
            len_b = len_v[pl.ds(b, LANES)][0]
            pltpu.make_async_copy(x_hbm.at[idx_v], x_v, sg).wait()

            def row_body(r, rc):
                p = p0 + r
                valid = jnp.logical_and(p >= 1, p <= len_b)
                m = jnp.where(valid, scale, jnp.float32(0.0))
                for k in range(NV):
                    sl = pl.ds(k * LANES, LANES)
                    x_v[r, sl] = x_v[r, sl] * m + pe_v[r, sl]
                return rc

            lax.fori_loop(0, CH, row_body, jnp.int32(0))

            @pl.when(p0 == 0)
            def _():
                for k in range(NV):
                    sl = pl.ds(k * LANES, LANES)
                    x_v[0, sl] = x_v[0, sl] + beg_v[sl]

            pltpu.async_copy(x_v, out_hbm.at[b, pl.ds(p0, CH)], ss)

        def drain_store(gpar):
            pltpu.make_async_copy(gbufs[gpar][0], out_hbm.at[0, pl.ds(0, CH)],
                                  gbufs[gpar][3]).wait()

        def slot_body(j, par):
            pe_v = pebufs[par]
            bl_v = blbufs[par]

            @pl.when(j + 1 < K)
            def _():
                issue_slot_loads(j + 1, 1 - par)

            pltpu.make_async_copy(pe_hbm.at[pl.ds(0, CH)], pe_v,
                                  lpsems[par]).wait()
            pltpu.make_async_copy(bl_hbm.at[0, 0], bl_v, lbsems[par]).wait()

            p0 = slot_p0(j)
            gcnt_c = bl_v[pl.ds(0, LANES)][0]
            pe_start = bl_v[pl.ds(1, LANES)][0]

            def pe_body(i, pc):
                b = get_b(bl_v, i)
                pltpu.async_copy(pe_hbm.at[pl.ds(p0, CH)],
                                 out_hbm.at[b, pl.ds(p0, CH)], s_pe)
                return pc

            lax.fori_loop(pe_start, B, pe_body, jnp.int32(0))

            @pl.when(gcnt_c > 0)
            def _():
                issue_gather(bl_v, p0, 0, 0)

            def _stage(i, gpar):
                nxt = 1 - gpar

                @pl.when(i + 1 < gcnt_c)
                def _():
                    @pl.when(i >= 1)
                    def _():
                        drain_store(nxt)

                    issue_gather(bl_v, p0, i + 1, nxt)

                compute_store(bl_v, pe_v, p0, i, gpar)

            def pipe_body(i, pc):
                @pl.when(i % 2 == 0)
                def _():
                    _stage(i, 0)

                @pl.when(i % 2 == 1)
                def _():
                    _stage(i, 1)

                return pc

            lax.fori_loop(0, gcnt_c, pipe_body, jnp.int32(0))

            for want in (2, 1):
                @pl.when(jnp.logical_and(gcnt_c >= want,
                                         (gcnt_c - want) % 2 == 0))
                def _():
                    drain_store(0)

                @pl.when(jnp.logical_and(gcnt_c >= want,
                                         (gcnt_c - want) % 2 == 1))
                def _():
                    drain_store(1)

            def pe_drain(i, pc):
                pltpu.make_async_copy(
                    pe_hbm.at[pl.ds(0, CH)], out_hbm.at[0, pl.ds(0, CH)],
                    s_pe).wait()
                return pc

            lax.fori_loop(pe_start, B, pe_drain, jnp.int32(0))

        issue_slot_loads(0, 0)

        def run_slot(j, carry):
            @pl.when(j % 2 == 0)
            def _():
                slot_body(j, 0)

            @pl.when(j % 2 == 1)
            def _():
                slot_body(j, 1)

            return carry

        lax.fori_loop(0, K, run_slot, jnp.int32(0))

    fn = pl.kernel(
        body,
        out_type=jax.ShapeDtypeStruct((B, ML, D), jnp.float32),
        mesh=mesh,
        scratch_types=[
            pltpu.VMEM((B + LANES,), jnp.int32),
            pltpu.VMEM((B + LANES,), jnp.int32),
            pltpu.VMEM((CH, D), jnp.float32),
            pltpu.VMEM((CH, D), jnp.float32),
            pltpu.VMEM((CH, D), jnp.float32),
            pltpu.VMEM((CH, D), jnp.float32),
            pltpu.VMEM((CH,), jnp.int32),
            pltpu.VMEM((CH,), jnp.int32),
            pltpu.VMEM((BW,), jnp.int32),
            pltpu.VMEM((BW,), jnp.int32),
            pltpu.VMEM((SW,), jnp.int32),
            pltpu.VMEM((D,), jnp.float32),
            pltpu.SemaphoreType.DMA,
            pltpu.SemaphoreType.DMA,
            pltpu.SemaphoreType.DMA,
            pltpu.SemaphoreType.DMA,
            pltpu.SemaphoreType.DMA,
            pltpu.SemaphoreType.DMA,
            pltpu.SemaphoreType.DMA,
            pltpu.SemaphoreType.DMA,
            pltpu.SemaphoreType.DMA,
        ],
    )
    return fn(input_embs, lengths, starts, beg_seq_param, pe_tab,
              sched_tab, desc)


# speedup vs baseline: 11.1128x; 11.1128x over previous
import math

import jax
import jax.numpy as jnp
import numpy as np
from jax import lax
from jax.experimental import pallas as pl
from jax.experimental.pallas import tpu as pltpu
from jax.experimental.pallas import tpu_sc as plsc

HIDDEN = 512
PAD_MULT = 128
EXTRA = 1
MAX_LEN = 2176
LANES = 16
CH = 64
BW = 48

_SCHED_LENGTHS = (2048, 512, 1024, 1536, 768, 1280, 896, 1152,
                  640, 1408, 1024, 704, 960, 832, 1088, 512)


def _sinusoidal_pos_emb(max_len: int, d: int) -> np.ndarray:
    pos = np.arange(max_len, dtype=np.float32)[:, None]
    i = np.arange(0, d, 2, dtype=np.float32)
    div = np.exp(-math.log(10000.0) * i / d)
    pe = np.zeros((max_len, d), dtype=np.float32)
    pe[:, 0::2] = np.sin(pos * div)
    pe[:, 1::2] = np.cos(pos * div)
    return pe


def _chunk_schedule(n_chunks: int, n_workers: int) -> np.ndarray:
    weights = []
    for c in range(n_chunks):
        g = sum(1 for L in _SCHED_LENGTHS if c * CH <= L)
        weights.append(16 + 2 * g)
    order = sorted(range(n_chunks), key=lambda c: -weights[c])
    loads = [0] * n_workers
    bins = [[] for _ in range(n_workers)]
    for c in order:
        w = min(range(n_workers), key=lambda i: (loads[i], len(bins[i])))
        bins[w].append(c)
        loads[w] += weights[c]
    k = max(len(b) for b in bins)
    sched = np.full((n_workers, 1, k + LANES), n_chunks, dtype=np.int32)
    for w, b in enumerate(bins):
        sched[w, 0, :len(b)] = b
    return sched


def kernel(input_embs, input_seq_lengths, beg_seq_param):
    T, D = input_embs.shape
    B = input_seq_lengths.shape[0]
    ML = MAX_LEN
    scale = jnp.float32(math.sqrt(D))

    mesh = plsc.VectorSubcoreMesh(core_axis_name="c", subcore_axis_name="s")
    NC, NS = mesh.num_cores, mesh.num_subcores
    NW = NC * NS
    n_chunks = ML // CH
    assert ML % CH == 0
    NV = D // LANES

    pe_tab = jnp.asarray(_sinusoidal_pos_emb(ML, D))
    sched_tab = jnp.asarray(_chunk_schedule(n_chunks, NW))
    SW = sched_tab.shape[2]
    K = SW - LANES

    lengths = input_seq_lengths.astype(jnp.int32)
    starts = jnp.concatenate([jnp.zeros((1,), jnp.int32),
                              jnp.cumsum(lengths)[:-1].astype(jnp.int32)])
    cond = (CH * jnp.arange(n_chunks, dtype=jnp.int32)[:, None]) <= lengths[None, :]
    gcnt = jnp.sum(cond, axis=1).astype(jnp.int32)
    ordr = jnp.argsort(jnp.logical_not(cond), axis=1, stable=True).astype(jnp.int32)
    desc = jnp.concatenate(
        [gcnt[:, None], gcnt[:, None], ordr,
         jnp.zeros((n_chunks, BW - 2 - B), jnp.int32)], axis=1)
    dummy = jnp.concatenate(
        [jnp.zeros((1, 1), jnp.int32), jnp.full((1, 1), B, jnp.int32),
         jnp.zeros((1, BW - 2), jnp.int32)], axis=1)
    desc = jnp.concatenate([desc, dummy], axis=0)[:, None, :]

    def body(x_hbm, len_hbm, st_hbm, beg_hbm, pe_hbm, sched_hbm, bl_hbm,
             out_hbm,
             len_v, st_v, pe0_v, xa_v, xb_v, idxa_v, idxb_v,
             bl0_v, bl1_v, schv, beg_v,
             sg_a, sg_b, ss_a, ss_b, s_pe, s_lb0, s_lb1):
        cid = lax.axis_index("c")
        sid = lax.axis_index("s")
        w = sid * NC + cid

        pltpu.sync_copy(len_hbm, len_v.at[pl.ds(0, B)])
        pltpu.sync_copy(st_hbm, st_v.at[pl.ds(0, B)])
        pltpu.sync_copy(sched_hbm.at[w, 0], schv)

        iota = lax.iota(jnp.int32, LANES)

        pltpu.sync_copy(beg_hbm, beg_v)
        for k in range(NV):
            sl = pl.ds(k * LANES, LANES)
            beg_v[sl] = beg_v[sl] * scale

        gbufs = ((xa_v, idxa_v, sg_a, ss_a), (xb_v, idxb_v, sg_b, ss_b))
        blbufs = (bl0_v, bl1_v)
        lbsems = (s_lb0, s_lb1)

        def slot_c(j):
            return schv[pl.ds(j, LANES)][0]

        def slot_p0(j):
            return jnp.minimum(slot_c(j), n_chunks - 1) * CH

        def issue_slot_loads(j, par):
            pltpu.async_copy(bl_hbm.at[slot_c(j), 0], blbufs[par],
                             lbsems[par])

        def get_b(bl_v, i):
            return bl_v[pl.ds(i + 2, LANES)][0]

        def issue_gather(bl_v, p0, i, gpar):
            x_v, idx_v, sg, _ = gbufs[gpar]
            b = get_b(bl_v, i)
            st_b = st_v[pl.ds(b, LANES)][0]
            base = st_b + p0 - 1
            for j in range(CH // LANES):
                idx_v[pl.ds(j * LANES, LANES)] = jnp.clip(
                    base + j * LANES + iota, 0, T - 1)
            pltpu.async_copy(x_hbm.at[idx_v], x_v, sg)

        def compute_store(bl_v, pe_v, p0, i, gpar):
            x_v, idx_v, sg, ss = gbufs[gpar]
            b = get_b(bl_v, i)
            len_b = len_v[pl.ds(b, LANES)][0]
            pltpu.make_async_copy(x_hbm.at[idx_v], x_v, sg).wait()

            def row_body(r, rc):
                p = p0 + r
                valid = jnp.logical_and(p >= 1, p <= len_b)
                m = jnp.where(valid, scale, jnp.float32(0.0))
                for k in range(NV):
                    sl = pl.ds(k * LANES, LANES)
                    x_v[r, sl] = x_v[r, sl] * m + pe_v[r, sl]
                return rc

            lax.fori_loop(0, CH, row_body, jnp.int32(0))

            @pl.when(p0 == 0)
            def _():
                for k in range(NV):
                    sl = pl.ds(k * LANES, LANES)
                    x_v[0, sl] = x_v[0, sl] + beg_v[sl]

            pltpu.async_copy(x_v, out_hbm.at[b, pl.ds(p0, CH)], ss)

        def drain_store(gpar):
            pltpu.make_async_copy(gbufs[gpar][0], out_hbm.at[0, pl.ds(0, CH)],
                                  gbufs[gpar][3]).wait()

        def slot_body(j, par):
            pe_v = pe0_v
            bl_v = blbufs[par]

            @pl.when(j + 1 < K)
            def _():
                issue_slot_loads(j + 1, 1 - par)

            pltpu.make_async_copy(bl_hbm.at[0, 0], bl_v, lbsems[par]).wait()

            p0 = slot_p0(j)
            pltpu.sync_copy(pe_hbm.at[pl.ds(p0, CH)], pe_v)
            gcnt_c = bl_v[pl.ds(0, LANES)][0]
            pe_start = bl_v[pl.ds(1, LANES)][0]

            def pe_body(i, pc):
                b = get_b(bl_v, i)
                pltpu.async_copy(pe_v, out_hbm.at[b, pl.ds(p0, CH)], s_pe)
                return pc

            lax.fori_loop(pe_start, B, pe_body, jnp.int32(0))

            @pl.when(gcnt_c > 0)
            def _():
                issue_gather(bl_v, p0, 0, 0)

            def _stage(i, gpar):
                nxt = 1 - gpar

                @pl.when(i + 1 < gcnt_c)
                def _():
                    @pl.when(i >= 1)
                    def _():
                        drain_store(nxt)

                    issue_gather(bl_v, p0, i + 1, nxt)

                compute_store(bl_v, pe_v, p0, i, gpar)

            def pipe_body(i, pc):
                @pl.when(i % 2 == 0)
                def _():
                    _stage(i, 0)

                @pl.when(i % 2 == 1)
                def _():
                    _stage(i, 1)

                return pc

            lax.fori_loop(0, gcnt_c, pipe_body, jnp.int32(0))

            for want in (2, 1):
                @pl.when(jnp.logical_and(gcnt_c >= want,
                                         (gcnt_c - want) % 2 == 0))
                def _():
                    drain_store(0)

                @pl.when(jnp.logical_and(gcnt_c >= want,
                                         (gcnt_c - want) % 2 == 1))
                def _():
                    drain_store(1)

            def pe_drain(i, pc):
                pltpu.make_async_copy(
                    pe_v, out_hbm.at[0, pl.ds(0, CH)], s_pe).wait()
                return pc

            lax.fori_loop(pe_start, B, pe_drain, jnp.int32(0))

        issue_slot_loads(0, 0)

        def run_slot(j, carry):
            @pl.when(j % 2 == 0)
            def _():
                slot_body(j, 0)

            @pl.when(j % 2 == 1)
            def _():
                slot_body(j, 1)

            return carry

        lax.fori_loop(0, K, run_slot, jnp.int32(0))

    fn = pl.kernel(
        body,
        out_type=jax.ShapeDtypeStruct((B, ML, D), jnp.float32),
        mesh=mesh,
        scratch_types=[
            pltpu.VMEM((B + LANES,), jnp.int32),
            pltpu.VMEM((B + LANES,), jnp.int32),
            pltpu.VMEM((CH, D), jnp.float32),
            pltpu.VMEM((CH, D), jnp.float32),
            pltpu.VMEM((CH, D), jnp.float32),
            pltpu.VMEM((CH,), jnp.int32),
            pltpu.VMEM((CH,), jnp.int32),
            pltpu.VMEM((BW,), jnp.int32),
            pltpu.VMEM((BW,), jnp.int32),
            pltpu.VMEM((SW,), jnp.int32),
            pltpu.VMEM((D,), jnp.float32),
            pltpu.SemaphoreType.DMA,
            pltpu.SemaphoreType.DMA,
            pltpu.SemaphoreType.DMA,
            pltpu.SemaphoreType.DMA,
            pltpu.SemaphoreType.DMA,
            pltpu.SemaphoreType.DMA,
            pltpu.SemaphoreType.DMA,
        ],
    )
    return fn(input_embs, lengths, starts, beg_seq_param, pe_tab,
              sched_tab, desc)


# speedup vs baseline: 12.9444x; 1.1648x over previous
import math

import jax
import jax.numpy as jnp
import numpy as np
from jax import lax
from jax.experimental import pallas as pl
from jax.experimental.pallas import tpu as pltpu
from jax.experimental.pallas import tpu_sc as plsc

HIDDEN = 512
PAD_MULT = 128
EXTRA = 1
MAX_LEN = 2176
LANES = 16
CH = 16
BW = 48

_SCHED_LENGTHS = (2048, 512, 1024, 1536, 768, 1280, 896, 1152,
                  640, 1408, 1024, 704, 960, 832, 1088, 512)


def _sinusoidal_pos_emb(max_len: int, d: int) -> np.ndarray:
    pos = np.arange(max_len, dtype=np.float32)[:, None]
    i = np.arange(0, d, 2, dtype=np.float32)
    div = np.exp(-math.log(10000.0) * i / d)
    pe = np.zeros((max_len, d), dtype=np.float32)
    pe[:, 0::2] = np.sin(pos * div)
    pe[:, 1::2] = np.cos(pos * div)
    return pe


def _chunk_schedule(n_chunks: int, n_workers: int) -> np.ndarray:
    weights = []
    for c in range(n_chunks):
        g = sum(1 for L in _SCHED_LENGTHS if c * CH <= L)
        weights.append(16 + 2 * g)
    order = sorted(range(n_chunks), key=lambda c: -weights[c])
    loads = [0] * n_workers
    bins = [[] for _ in range(n_workers)]
    for c in order:
        w = min(range(n_workers), key=lambda i: (loads[i], len(bins[i])))
        bins[w].append(c)
        loads[w] += weights[c]
    k = max(len(b) for b in bins)
    sched = np.full((n_workers, 1, k + LANES), n_chunks, dtype=np.int32)
    for w, b in enumerate(bins):
        sched[w, 0, :len(b)] = b
    return sched


def kernel(input_embs, input_seq_lengths, beg_seq_param):
    T, D = input_embs.shape
    B = input_seq_lengths.shape[0]
    ML = MAX_LEN
    scale = jnp.float32(math.sqrt(D))

    mesh = plsc.VectorSubcoreMesh(core_axis_name="c", subcore_axis_name="s")
    NC, NS = mesh.num_cores, mesh.num_subcores
    NW = NC * NS
    n_chunks = ML // CH
    assert ML % CH == 0
    NV = D // LANES

    pe_tab = jnp.asarray(_sinusoidal_pos_emb(ML, D))
    sched_tab = jnp.asarray(_chunk_schedule(n_chunks, NW))
    SW = sched_tab.shape[2]
    K = SW - LANES

    lengths = input_seq_lengths.astype(jnp.int32)
    starts = jnp.concatenate([jnp.zeros((1,), jnp.int32),
                              jnp.cumsum(lengths)[:-1].astype(jnp.int32)])
    cond = (CH * jnp.arange(n_chunks, dtype=jnp.int32)[:, None]) <= lengths[None, :]
    gcnt = jnp.sum(cond, axis=1).astype(jnp.int32)
    ordr = jnp.argsort(jnp.logical_not(cond), axis=1, stable=True).astype(jnp.int32)
    desc = jnp.concatenate(
        [gcnt[:, None], gcnt[:, None], ordr,
         jnp.zeros((n_chunks, BW - 2 - B), jnp.int32)], axis=1)
    dummy = jnp.concatenate(
        [jnp.zeros((1, 1), jnp.int32), jnp.full((1, 1), B, jnp.int32),
         jnp.zeros((1, BW - 2), jnp.int32)], axis=1)
    desc = jnp.concatenate([desc, dummy], axis=0)[:, None, :]

    def body(x_hbm, len_hbm, st_hbm, beg_hbm, pe_hbm, sched_hbm, bl_hbm,
             out_hbm,
             len_v, st_v, pe0_v, pe1_v, xa_v, xb_v, idxa_v, idxb_v,
             bl0_v, bl1_v, schv, beg_v,
             sg_a, sg_b, ss_a, ss_b, s_pe, s_lp0, s_lp1, s_lb0, s_lb1):
        cid = lax.axis_index("c")
        sid = lax.axis_index("s")
        w = sid * NC + cid

        pltpu.sync_copy(len_hbm, len_v.at[pl.ds(0, B)])
        pltpu.sync_copy(st_hbm, st_v.at[pl.ds(0, B)])
        pltpu.sync_copy(sched_hbm.at[w, 0], schv)

        iota = lax.iota(jnp.int32, LANES)

        pltpu.sync_copy(beg_hbm, beg_v)
        for k in range(NV):
            sl = pl.ds(k * LANES, LANES)
            beg_v[sl] = beg_v[sl] * scale

        gbufs = ((xa_v, idxa_v, sg_a, ss_a), (xb_v, idxb_v, sg_b, ss_b))
        pebufs = (pe0_v, pe1_v)
        blbufs = (bl0_v, bl1_v)
        lpsems = (s_lp0, s_lp1)
        lbsems = (s_lb0, s_lb1)

        def slot_c(j):
            return schv[pl.ds(j, LANES)][0]

        def slot_p0(j):
            return jnp.minimum(slot_c(j), n_chunks - 1) * CH

        def issue_slot_loads(j, par):
            c = slot_c(j)
            pltpu.async_copy(pe_hbm.at[pl.ds(slot_p0(j), CH)], pebufs[par],
                             lpsems[par])
            pltpu.async_copy(bl_hbm.at[c, 0], blbufs[par], lbsems[par])

        def get_b(bl_v, i):
            return bl_v[pl.ds(i + 2, LANES)][0]

        def issue_gather(bl_v, p0, i, gpar):
            x_v, idx_v, sg, _ = gbufs[gpar]
            b = get_b(bl_v, i)
            st_b = st_v[pl.ds(b, LANES)][0]
            base = st_b + p0 - 1
            for j in range(CH // LANES):
                idx_v[pl.ds(j * LANES, LANES)] = jnp.clip(
                    base + j * LANES + iota, 0, T - 1)
            pltpu.async_copy(x_hbm.at[idx_v], x_v, sg)

        def compute_store(bl_v, pe_v, p0, i, gpar):
            x_v, idx_v, sg, ss = gbufs[gpar]
            b = get_b(bl_v, i)
            len_b = len_v[pl.ds(b, LANES)][0]
            pltpu.make_async_copy(x_hbm.at[idx_v], x_v, sg).wait()

            def row_body(r, rc):
                p = p0 + r
                valid = jnp.logical_and(p >= 1, p <= len_b)
                m = jnp.where(valid, scale, jnp.float32(0.0))
                for k in range(NV):
                    sl = pl.ds(k * LANES, LANES)
                    x_v[r, sl] = x_v[r, sl] * m + pe_v[r, sl]
                return rc

            lax.fori_loop(0, CH, row_body, jnp.int32(0))

            @pl.when(p0 == 0)
            def _():
                for k in range(NV):
                    sl = pl.ds(k * LANES, LANES)
                    x_v[0, sl] = x_v[0, sl] + beg_v[sl]

            pltpu.async_copy(x_v, out_hbm.at[b, pl.ds(p0, CH)], ss)

        def drain_store(gpar):
            pltpu.make_async_copy(gbufs[gpar][0], out_hbm.at[0, pl.ds(0, CH)],
                                  gbufs[gpar][3]).wait()

        def slot_body(j, par):
            pe_v = pebufs[par]
            bl_v = blbufs[par]

            @pl.when(j + 1 < K)
            def _():
                issue_slot_loads(j + 1, 1 - par)

            pltpu.make_async_copy(pe_hbm.at[pl.ds(0, CH)], pe_v,
                                  lpsems[par]).wait()
            pltpu.make_async_copy(bl_hbm.at[0, 0], bl_v, lbsems[par]).wait()

            p0 = slot_p0(j)
            gcnt_c = bl_v[pl.ds(0, LANES)][0]
            pe_start = bl_v[pl.ds(1, LANES)][0]

            def pe_body(i, pc):
                b = get_b(bl_v, i)
                pltpu.async_copy(pe_v, out_hbm.at[b, pl.ds(p0, CH)], s_pe)
                return pc

            lax.fori_loop(pe_start, B, pe_body, jnp.int32(0))

            @pl.when(gcnt_c > 0)
            def _():
                issue_gather(bl_v, p0, 0, 0)

            def _stage(i, gpar):
                nxt = 1 - gpar

                @pl.when(i + 1 < gcnt_c)
                def _():
                    @pl.when(i >= 1)
                    def _():
                        drain_store(nxt)

                    issue_gather(bl_v, p0, i + 1, nxt)

                compute_store(bl_v, pe_v, p0, i, gpar)

            def pipe_body(i, pc):
                @pl.when(i % 2 == 0)
                def _():
                    _stage(i, 0)

                @pl.when(i % 2 == 1)
                def _():
                    _stage(i, 1)

                return pc

            lax.fori_loop(0, gcnt_c, pipe_body, jnp.int32(0))

            for want in (2, 1):
                @pl.when(jnp.logical_and(gcnt_c >= want,
                                         (gcnt_c - want) % 2 == 0))
                def _():
                    drain_store(0)

                @pl.when(jnp.logical_and(gcnt_c >= want,
                                         (gcnt_c - want) % 2 == 1))
                def _():
                    drain_store(1)

            def pe_drain(i, pc):
                pltpu.make_async_copy(
                    pe_v, out_hbm.at[0, pl.ds(0, CH)], s_pe).wait()
                return pc

            lax.fori_loop(pe_start, B, pe_drain, jnp.int32(0))

        issue_slot_loads(0, 0)

        def run_slot(j, carry):
            @pl.when(j % 2 == 0)
            def _():
                slot_body(j, 0)

            @pl.when(j % 2 == 1)
            def _():
                slot_body(j, 1)

            return carry

        lax.fori_loop(0, K, run_slot, jnp.int32(0))

    fn = pl.kernel(
        body,
        out_type=jax.ShapeDtypeStruct((B, ML, D), jnp.float32),
        mesh=mesh,
        scratch_types=[
            pltpu.VMEM((B + LANES,), jnp.int32),
            pltpu.VMEM((B + LANES,), jnp.int32),
            pltpu.VMEM((CH, D), jnp.float32),
            pltpu.VMEM((CH, D), jnp.float32),
            pltpu.VMEM((CH, D), jnp.float32),
            pltpu.VMEM((CH, D), jnp.float32),
            pltpu.VMEM((CH,), jnp.int32),
            pltpu.VMEM((CH,), jnp.int32),
            pltpu.VMEM((BW,), jnp.int32),
            pltpu.VMEM((BW,), jnp.int32),
            pltpu.VMEM((SW,), jnp.int32),
            pltpu.VMEM((D,), jnp.float32),
            pltpu.SemaphoreType.DMA,
            pltpu.SemaphoreType.DMA,
            pltpu.SemaphoreType.DMA,
            pltpu.SemaphoreType.DMA,
            pltpu.SemaphoreType.DMA,
            pltpu.SemaphoreType.DMA,
            pltpu.SemaphoreType.DMA,
            pltpu.SemaphoreType.DMA,
            pltpu.SemaphoreType.DMA,
        ],
    )
    return fn(input_embs, lengths, starts, beg_seq_param, pe_tab,
              sched_tab, desc)


# speedup vs baseline: 13.7126x; 1.0593x over previous
import math

import jax
import jax.numpy as jnp
import numpy as np
from jax import lax
from jax.experimental import pallas as pl
from jax.experimental.pallas import tpu as pltpu
from jax.experimental.pallas import tpu_sc as plsc

HIDDEN = 512
PAD_MULT = 128
EXTRA = 1
MAX_LEN = 2176
LANES = 16
CH = 32
BW = 48

_SCHED_LENGTHS = (2048, 512, 1024, 1536, 768, 1280, 896, 1152,
                  640, 1408, 1024, 704, 960, 832, 1088, 512)


def _sinusoidal_pos_emb(max_len: int, d: int) -> np.ndarray:
    pos = np.arange(max_len, dtype=np.float32)[:, None]
    i = np.arange(0, d, 2, dtype=np.float32)
    div = np.exp(-math.log(10000.0) * i / d)
    pe = np.zeros((max_len, d), dtype=np.float32)
    pe[:, 0::2] = np.sin(pos * div)
    pe[:, 1::2] = np.cos(pos * div)
    return pe


def _chunk_schedule(n_chunks: int, n_workers: int) -> np.ndarray:
    weights = []
    for c in range(n_chunks):
        g = sum(1 for L in _SCHED_LENGTHS if c * CH <= L)
        weights.append(16 + g)
    order = sorted(range(n_chunks), key=lambda c: -weights[c])
    loads = [0] * n_workers
    bins = [[] for _ in range(n_workers)]
    for c in order:
        w = min(range(n_workers), key=lambda i: (loads[i], len(bins[i])))
        bins[w].append(c)
        loads[w] += weights[c]
    k = max(len(b) for b in bins)
    sched = np.full((n_workers, 1, k + LANES), n_chunks, dtype=np.int32)
    for w, b in enumerate(bins):
        sched[w, 0, :len(b)] = b
    return sched


def kernel(input_embs, input_seq_lengths, beg_seq_param):
    T, D = input_embs.shape
    B = input_seq_lengths.shape[0]
    ML = MAX_LEN
    scale = jnp.float32(math.sqrt(D))

    mesh = plsc.VectorSubcoreMesh(core_axis_name="c", subcore_axis_name="s")
    NC, NS = mesh.num_cores, mesh.num_subcores
    NW = NC * NS
    n_chunks = ML // CH
    assert ML % CH == 0
    NV = D // LANES

    pe_tab = jnp.asarray(_sinusoidal_pos_emb(ML, D))
    sched_tab = jnp.asarray(_chunk_schedule(n_chunks, NW))
    SW = sched_tab.shape[2]
    K = SW - LANES

    lengths = input_seq_lengths.astype(jnp.int32)
    starts = jnp.concatenate([jnp.zeros((1,), jnp.int32),
                              jnp.cumsum(lengths)[:-1].astype(jnp.int32)])
    cond = (CH * jnp.arange(n_chunks, dtype=jnp.int32)[:, None]) <= lengths[None, :]
    gcnt = jnp.sum(cond, axis=1).astype(jnp.int32)
    ordr = jnp.argsort(jnp.logical_not(cond), axis=1, stable=True).astype(jnp.int32)
    desc = jnp.concatenate(
        [gcnt[:, None], gcnt[:, None], ordr,
         jnp.zeros((n_chunks, BW - 2 - B), jnp.int32)], axis=1)
    dummy = jnp.concatenate(
        [jnp.zeros((1, 1), jnp.int32), jnp.full((1, 1), B, jnp.int32),
         jnp.zeros((1, BW - 2), jnp.int32)], axis=1)
    desc = jnp.concatenate([desc, dummy], axis=0)[:, None, :]

    def body(x_hbm, len_hbm, st_hbm, beg_hbm, pe_hbm, sched_hbm, bl_hbm,
             out_hbm,
             len_v, st_v, pe0_v, pe1_v, xa_v, xb_v, idxa_v, idxb_v,
             bl0_v, bl1_v, schv, beg_v,
             sg_a, sg_b, ss_a, ss_b, s_pe, s_lp0, s_lp1, s_lb0, s_lb1):
        cid = lax.axis_index("c")
        sid = lax.axis_index("s")
        w = sid * NC + cid

        pltpu.sync_copy(len_hbm, len_v.at[pl.ds(0, B)])
        pltpu.sync_copy(st_hbm, st_v.at[pl.ds(0, B)])
        pltpu.sync_copy(sched_hbm.at[w, 0], schv)

        iota = lax.iota(jnp.int32, LANES)

        pltpu.sync_copy(beg_hbm, beg_v)
        for k in range(NV):
            sl = pl.ds(k * LANES, LANES)
            beg_v[sl] = beg_v[sl] * scale

        gbufs = ((xa_v, idxa_v, sg_a, ss_a), (xb_v, idxb_v, sg_b, ss_b))
        pebufs = (pe0_v, pe1_v)
        blbufs = (bl0_v, bl1_v)
        lpsems = (s_lp0, s_lp1)
        lbsems = (s_lb0, s_lb1)

        def slot_c(j):
            return schv[pl.ds(j, LANES)][0]

        def slot_p0(j):
            return jnp.minimum(slot_c(j), n_chunks - 1) * CH

        def issue_slot_loads(j, par):
            c = slot_c(j)
            pltpu.async_copy(pe_hbm.at[pl.ds(slot_p0(j), CH)], pebufs[par],
                             lpsems[par])
            pltpu.async_copy(bl_hbm.at[c, 0], blbufs[par], lbsems[par])

        def get_b(bl_v, i):
            return bl_v[pl.ds(i + 2, LANES)][0]

        def issue_gather(bl_v, p0, i, gpar):
            x_v, idx_v, sg, _ = gbufs[gpar]
            b = get_b(bl_v, i)
            st_b = st_v[pl.ds(b, LANES)][0]
            base = st_b + p0 - 1
            for j in range(CH // LANES):
                idx_v[pl.ds(j * LANES, LANES)] = jnp.clip(
                    base + j * LANES + iota, 0, T - 1)
            pltpu.async_copy(x_hbm.at[idx_v], x_v, sg)

        def compute_store(bl_v, pe_v, p0, i, gpar):
            x_v, idx_v, sg, ss = gbufs[gpar]
            b = get_b(bl_v, i)
            len_b = len_v[pl.ds(b, LANES)][0]
            pltpu.make_async_copy(x_hbm.at[idx_v], x_v, sg).wait()

            def row_body(r, rc):
                p = p0 + r
                valid = jnp.logical_and(p >= 1, p <= len_b)
                m = jnp.where(valid, scale, jnp.float32(0.0))
                for k in range(NV):
                    sl = pl.ds(k * LANES, LANES)
                    x_v[r, sl] = x_v[r, sl] * m + pe_v[r, sl]
                return rc

            lax.fori_loop(0, CH, row_body, jnp.int32(0))

            @pl.when(p0 == 0)
            def _():
                for k in range(NV):
                    sl = pl.ds(k * LANES, LANES)
                    x_v[0, sl] = x_v[0, sl] + beg_v[sl]

            pltpu.async_copy(x_v, out_hbm.at[b, pl.ds(p0, CH)], ss)

        def drain_store(gpar):
            pltpu.make_async_copy(gbufs[gpar][0], out_hbm.at[0, pl.ds(0, CH)],
                                  gbufs[gpar][3]).wait()

        def slot_body(j, par):
            pe_v = pebufs[par]
            bl_v = blbufs[par]

            @pl.when(j + 1 < K)
            def _():
                issue_slot_loads(j + 1, 1 - par)

            pltpu.make_async_copy(pe_hbm.at[pl.ds(0, CH)], pe_v,
                                  lpsems[par]).wait()
            pltpu.make_async_copy(bl_hbm.at[0, 0], bl_v, lbsems[par]).wait()

            p0 = slot_p0(j)
            gcnt_c = bl_v[pl.ds(0, LANES)][0]
            pe_start = bl_v[pl.ds(1, LANES)][0]

            def pe_body(i, pc):
                b = get_b(bl_v, i)
                pltpu.async_copy(pe_v, out_hbm.at[b, pl.ds(p0, CH)], s_pe)
                return pc

            lax.fori_loop(pe_start, B, pe_body, jnp.int32(0))

            @pl.when(gcnt_c > 0)
            def _():
                issue_gather(bl_v, p0, 0, 0)

            def _stage(i, gpar):
                nxt = 1 - gpar

                @pl.when(i + 1 < gcnt_c)
                def _():
                    @pl.when(i >= 1)
                    def _():
                        drain_store(nxt)

                    issue_gather(bl_v, p0, i + 1, nxt)

                compute_store(bl_v, pe_v, p0, i, gpar)

            def pipe_body(i, pc):
                @pl.when(i % 2 == 0)
                def _():
                    _stage(i, 0)

                @pl.when(i % 2 == 1)
                def _():
                    _stage(i, 1)

                return pc

            lax.fori_loop(0, gcnt_c, pipe_body, jnp.int32(0))

            for want in (2, 1):
                @pl.when(jnp.logical_and(gcnt_c >= want,
                                         (gcnt_c - want) % 2 == 0))
                def _():
                    drain_store(0)

                @pl.when(jnp.logical_and(gcnt_c >= want,
                                         (gcnt_c - want) % 2 == 1))
                def _():
                    drain_store(1)

            def pe_drain(i, pc):
                pltpu.make_async_copy(
                    pe_v, out_hbm.at[0, pl.ds(0, CH)], s_pe).wait()
                return pc

            lax.fori_loop(pe_start, B, pe_drain, jnp.int32(0))

        issue_slot_loads(0, 0)

        def run_slot(j, carry):
            @pl.when(j % 2 == 0)
            def _():
                slot_body(j, 0)

            @pl.when(j % 2 == 1)
            def _():
                slot_body(j, 1)

            return carry

        lax.fori_loop(0, K, run_slot, jnp.int32(0))

    fn = pl.kernel(
        body,
        out_type=jax.ShapeDtypeStruct((B, ML, D), jnp.float32),
        mesh=mesh,
        scratch_types=[
            pltpu.VMEM((B + LANES,), jnp.int32),
            pltpu.VMEM((B + LANES,), jnp.int32),
            pltpu.VMEM((CH, D), jnp.float32),
            pltpu.VMEM((CH, D), jnp.float32),
            pltpu.VMEM((CH, D), jnp.float32),
            pltpu.VMEM((CH, D), jnp.float32),
            pltpu.VMEM((CH,), jnp.int32),
            pltpu.VMEM((CH,), jnp.int32),
            pltpu.VMEM((BW,), jnp.int32),
            pltpu.VMEM((BW,), jnp.int32),
            pltpu.VMEM((SW,), jnp.int32),
            pltpu.VMEM((D,), jnp.float32),
            pltpu.SemaphoreType.DMA,
            pltpu.SemaphoreType.DMA,
            pltpu.SemaphoreType.DMA,
            pltpu.SemaphoreType.DMA,
            pltpu.SemaphoreType.DMA,
            pltpu.SemaphoreType.DMA,
            pltpu.SemaphoreType.DMA,
            pltpu.SemaphoreType.DMA,
            pltpu.SemaphoreType.DMA,
        ],
    )
    return fn(input_embs, lengths, starts, beg_seq_param, pe_tab,
              sched_tab, desc)


# speedup vs baseline: 14.4347x; 1.0527x over previous
import math

import jax
import jax.numpy as jnp
import numpy as np
from jax import lax
from jax.experimental import pallas as pl
from jax.experimental.pallas import tpu as pltpu
from jax.experimental.pallas import tpu_sc as plsc

HIDDEN = 512
PAD_MULT = 128
EXTRA = 1
MAX_LEN = 2176
LANES = 16
CH = 32
BW = 48

_SCHED_LENGTHS = (2048, 512, 1024, 1536, 768, 1280, 896, 1152,
                  640, 1408, 1024, 704, 960, 832, 1088, 512)


def _sinusoidal_pos_emb(max_len: int, d: int) -> np.ndarray:
    pos = np.arange(max_len, dtype=np.float32)[:, None]
    i = np.arange(0, d, 2, dtype=np.float32)
    div = np.exp(-math.log(10000.0) * i / d)
    pe = np.zeros((max_len, d), dtype=np.float32)
    pe[:, 0::2] = np.sin(pos * div)
    pe[:, 1::2] = np.cos(pos * div)
    return pe


def _chunk_schedule(n_chunks: int, n_workers: int) -> np.ndarray:
    weights = []
    for c in range(n_chunks):
        g = sum(1 for L in _SCHED_LENGTHS if c * CH <= L)
        weights.append(16 + g)
    order = sorted(range(n_chunks), key=lambda c: -weights[c])
    loads = [0] * n_workers
    bins = [[] for _ in range(n_workers)]
    for c in order:
        w = min(range(n_workers), key=lambda i: (loads[i], len(bins[i])))
        bins[w].append(c)
        loads[w] += weights[c]
    k = max(len(b) for b in bins)
    sched = np.full((n_workers, 1, k + LANES), n_chunks, dtype=np.int32)
    for w, b in enumerate(bins):
        sched[w, 0, :len(b)] = b
    return sched


def kernel(input_embs, input_seq_lengths, beg_seq_param):
    T, D = input_embs.shape
    B = input_seq_lengths.shape[0]
    ML = MAX_LEN
    scale = jnp.float32(math.sqrt(D))

    mesh = plsc.VectorSubcoreMesh(core_axis_name="c", subcore_axis_name="s")
    NC, NS = mesh.num_cores, mesh.num_subcores
    NW = NC * NS
    n_chunks = ML // CH
    assert ML % CH == 0
    NV = D // LANES

    pe_tab = jnp.asarray(_sinusoidal_pos_emb(ML, D))
    sched_tab = jnp.asarray(_chunk_schedule(n_chunks, NW))
    SW = sched_tab.shape[2]
    K = SW - LANES

    lengths = input_seq_lengths.astype(jnp.int32)
    starts = jnp.concatenate([jnp.zeros((1,), jnp.int32),
                              jnp.cumsum(lengths)[:-1].astype(jnp.int32)])
    cond = (CH * jnp.arange(n_chunks, dtype=jnp.int32)[:, None]) <= lengths[None, :]
    gcnt = jnp.sum(cond, axis=1).astype(jnp.int32)
    ordr = jnp.argsort(jnp.logical_not(cond), axis=1, stable=True).astype(jnp.int32)
    desc = jnp.concatenate(
        [gcnt[:, None], gcnt[:, None], ordr,
         jnp.zeros((n_chunks, BW - 2 - B), jnp.int32)], axis=1)
    dummy = jnp.concatenate(
        [jnp.zeros((1, 1), jnp.int32), jnp.full((1, 1), B, jnp.int32),
         jnp.zeros((1, BW - 2), jnp.int32)], axis=1)
    desc = jnp.concatenate([desc, dummy], axis=0)[:, None, :]

    def body(x_hbm, len_hbm, st_hbm, beg_hbm, pe_hbm, sched_hbm, bl_hbm,
             out_hbm,
             len_v, st_v, pe0_v, pe1_v, xa_v, xb_v, xc_v,
             idxa_v, idxb_v, idxc_v,
             bl0_v, bl1_v, schv, beg_v,
             sg_a, sg_b, sg_c, ss_a, ss_b, ss_c,
             s_pe, s_lp0, s_lp1, s_lb0, s_lb1):
        cid = lax.axis_index("c")
        sid = lax.axis_index("s")
        w = sid * NC + cid

        pltpu.sync_copy(len_hbm, len_v.at[pl.ds(0, B)])
        pltpu.sync_copy(st_hbm, st_v.at[pl.ds(0, B)])
        pltpu.sync_copy(sched_hbm.at[w, 0], schv)

        iota = lax.iota(jnp.int32, LANES)

        pltpu.sync_copy(beg_hbm, beg_v)
        for k in range(NV):
            sl = pl.ds(k * LANES, LANES)
            beg_v[sl] = beg_v[sl] * scale

        gbufs = ((xa_v, idxa_v, sg_a, ss_a), (xb_v, idxb_v, sg_b, ss_b),
                 (xc_v, idxc_v, sg_c, ss_c))
        pebufs = (pe0_v, pe1_v)
        blbufs = (bl0_v, bl1_v)
        lpsems = (s_lp0, s_lp1)
        lbsems = (s_lb0, s_lb1)

        def slot_c(j):
            return schv[pl.ds(j, LANES)][0]

        def slot_p0(j):
            return jnp.minimum(slot_c(j), n_chunks - 1) * CH

        def issue_slot_loads(j, par):
            c = slot_c(j)
            pltpu.async_copy(pe_hbm.at[pl.ds(slot_p0(j), CH)], pebufs[par],
                             lpsems[par])
            pltpu.async_copy(bl_hbm.at[c, 0], blbufs[par], lbsems[par])

        def get_b(bl_v, i):
            return bl_v[pl.ds(i + 2, LANES)][0]

        def issue_gather(bl_v, p0, i, gpar):
            x_v, idx_v, sg, _ = gbufs[gpar]
            b = get_b(bl_v, i)
            st_b = st_v[pl.ds(b, LANES)][0]
            base = st_b + p0 - 1
            for j in range(CH // LANES):
                idx_v[pl.ds(j * LANES, LANES)] = jnp.clip(
                    base + j * LANES + iota, 0, T - 1)
            pltpu.async_copy(x_hbm.at[idx_v], x_v, sg)

        def compute_store(bl_v, pe_v, p0, i, gpar):
            x_v, idx_v, sg, ss = gbufs[gpar]
            b = get_b(bl_v, i)
            len_b = len_v[pl.ds(b, LANES)][0]
            pltpu.make_async_copy(x_hbm.at[idx_v], x_v, sg).wait()

            def row_body(r, rc):
                p = p0 + r
                valid = jnp.logical_and(p >= 1, p <= len_b)
                m = jnp.where(valid, scale, jnp.float32(0.0))
                for k in range(NV):
                    sl = pl.ds(k * LANES, LANES)
                    x_v[r, sl] = x_v[r, sl] * m + pe_v[r, sl]
                return rc

            lax.fori_loop(0, CH, row_body, jnp.int32(0))

            @pl.when(p0 == 0)
            def _():
                for k in range(NV):
                    sl = pl.ds(k * LANES, LANES)
                    x_v[0, sl] = x_v[0, sl] + beg_v[sl]

            pltpu.async_copy(x_v, out_hbm.at[b, pl.ds(p0, CH)], ss)

        def drain_store(gpar):
            pltpu.make_async_copy(gbufs[gpar][0], out_hbm.at[0, pl.ds(0, CH)],
                                  gbufs[gpar][3]).wait()

        def slot_body(j, par):
            pe_v = pebufs[par]
            bl_v = blbufs[par]

            @pl.when(j + 1 < K)
            def _():
                issue_slot_loads(j + 1, 1 - par)

            pltpu.make_async_copy(pe_hbm.at[pl.ds(0, CH)], pe_v,
                                  lpsems[par]).wait()
            pltpu.make_async_copy(bl_hbm.at[0, 0], bl_v, lbsems[par]).wait()

            p0 = slot_p0(j)
            gcnt_c = bl_v[pl.ds(0, LANES)][0]
            pe_start = bl_v[pl.ds(1, LANES)][0]

            def pe_body(i, pc):
                b = get_b(bl_v, i)
                pltpu.async_copy(pe_v, out_hbm.at[b, pl.ds(p0, CH)], s_pe)
                return pc

            lax.fori_loop(pe_start, B, pe_body, jnp.int32(0))

            @pl.when(gcnt_c > 0)
            def _():
                issue_gather(bl_v, p0, 0, 0)

            def _stage(i, gpar):
                nxt = (gpar + 1) % 3

                @pl.when(i + 1 < gcnt_c)
                def _():
                    @pl.when(i >= 2)
                    def _():
                        drain_store(nxt)

                    issue_gather(bl_v, p0, i + 1, nxt)

                compute_store(bl_v, pe_v, p0, i, gpar)

            def pipe_body(i, pc):
                for par in range(3):
                    @pl.when(i % 3 == par)
                    def _(par=par):
                        _stage(i, par)

                return pc

            lax.fori_loop(0, gcnt_c, pipe_body, jnp.int32(0))

            for want in (3, 2, 1):
                for par in range(3):
                    @pl.when(jnp.logical_and(gcnt_c >= want,
                                             (gcnt_c - want) % 3 == par))
                    def _(par=par):
                        drain_store(par)

            def pe_drain(i, pc):
                pltpu.make_async_copy(
                    pe_v, out_hbm.at[0, pl.ds(0, CH)], s_pe).wait()
                return pc

            lax.fori_loop(pe_start, B, pe_drain, jnp.int32(0))

        issue_slot_loads(0, 0)

        def run_slot(j, carry):
            @pl.when(j % 2 == 0)
            def _():
                slot_body(j, 0)

            @pl.when(j % 2 == 1)
            def _():
                slot_body(j, 1)

            return carry

        lax.fori_loop(0, K, run_slot, jnp.int32(0))

    fn = pl.kernel(
        body,
        out_type=jax.ShapeDtypeStruct((B, ML, D), jnp.float32),
        mesh=mesh,
        scratch_types=[
            pltpu.VMEM((B + LANES,), jnp.int32),
            pltpu.VMEM((B + LANES,), jnp.int32),
            pltpu.VMEM((CH, D), jnp.float32),
            pltpu.VMEM((CH, D), jnp.float32),
            pltpu.VMEM((CH, D), jnp.float32),
            pltpu.VMEM((CH, D), jnp.float32),
            pltpu.VMEM((CH, D), jnp.float32),
            pltpu.VMEM((CH,), jnp.int32),
            pltpu.VMEM((CH,), jnp.int32),
            pltpu.VMEM((CH,), jnp.int32),
            pltpu.VMEM((BW,), jnp.int32),
            pltpu.VMEM((BW,), jnp.int32),
            pltpu.VMEM((SW,), jnp.int32),
            pltpu.VMEM((D,), jnp.float32),
            pltpu.SemaphoreType.DMA,
            pltpu.SemaphoreType.DMA,
            pltpu.SemaphoreType.DMA,
            pltpu.SemaphoreType.DMA,
            pltpu.SemaphoreType.DMA,
            pltpu.SemaphoreType.DMA,
            pltpu.SemaphoreType.DMA,
            pltpu.SemaphoreType.DMA,
            pltpu.SemaphoreType.DMA,
            pltpu.SemaphoreType.DMA,
            pltpu.SemaphoreType.DMA,
        ],
    )
    return fn(input_embs, lengths, starts, beg_seq_param, pe_tab,
              sched_tab, desc)
